# baseline (speedup 1.0000x reference)
import functools

import jax
import jax.numpy as jnp
from jax import lax
from jax.experimental import pallas as pl
from jax.experimental.pallas import tpu as pltpu
from jax.experimental.pallas import tpu_sc as plsc

N, D = 16384, 512
LO_MID, HI_MID = 30, 60

_info = plsc.get_sparse_core_info()
NC, NS, L = _info.num_cores, _info.num_subcores, _info.num_lanes
NW = NC * NS

N_SC = 5120
N_TC = N - N_SC
RPW = N_SC // NW
CH = 32
NCH = RPW // CH
JD = D // L

BR = 1024
G_TC = N_TC // BR


def _sc_partials(w_mean, ages, anchors):
    mesh = plsc.VectorSubcoreMesh(core_axis_name="c", subcore_axis_name="s")

    @functools.partial(
        pl.kernel,
        mesh=mesh,
        out_type=jax.ShapeDtypeStruct((NW, L), jnp.float32),
        scratch_types=[
            pltpu.VMEM((2, CH, D), jnp.float32),
            pltpu.VMEM((RPW,), jnp.float32),
            pltpu.VMEM((RPW,), jnp.int32),
            pltpu.VMEM((2, D), jnp.float32),
            pltpu.VMEM((D,), jnp.float32),
            pltpu.VMEM((L,), jnp.float32),
            pltpu.SemaphoreType.DMA,
            pltpu.SemaphoreType.DMA,
        ],
    )
    def k(w_hbm, ages_hbm, anch_hbm, out_hbm,
          wbuf, mval, agev, anch, dd, accv, sem0, sem1):
        cid = lax.axis_index("c")
        sid = lax.axis_index("s")
        wid = sid * NC + cid
        base = N_TC + wid * RPW

        pltpu.sync_copy(ages_hbm.at[pl.ds(base, RPW)], agev)
        pltpu.sync_copy(anch_hbm, anch)

        def prep_m(g, carry):
            a16 = agev[pl.ds(g * L, L)]
            d0 = jnp.abs(a16 - LO_MID)
            d1 = jnp.abs(a16 - HI_MID)
            mval[pl.ds(g * L, L)] = jnp.where(d1 < d0, 1.0, 0.0).astype(jnp.float32)
            return carry

        lax.fori_loop(0, RPW // L, prep_m, 0)

        def prep_dd(j, carry):
            dd[pl.ds(j * L, L)] = anch[1, pl.ds(j * L, L)] - anch[0, pl.ds(j * L, L)]
            return carry

        lax.fori_loop(0, JD, prep_dd, 0)

        sems = (sem0, sem1)

        def start(c, b):
            return pltpu.async_copy(
                w_hbm.at[pl.ds(base + c * CH, CH)], wbuf.at[b], sems[b])

        h = start(0, 0)
        acc = jnp.zeros((L,), jnp.float32)
        for c in range(NCH):
            b = c % 2
            h_next = start(c + 1, 1 - b) if c + 1 < NCH else None
            h.wait()

            def col_body(j, acc, c=c, b=b):
                a0 = anch[0, pl.ds(j * L, L)]
                dv = dd[pl.ds(j * L, L)]

                def grp_body(g, acc):
                    mv = mval[pl.ds(c * CH + g * L, L)]
                    for kk in range(L):
                        w = wbuf[b, g * L + kk, pl.ds(j * L, L)]
                        mb = jnp.broadcast_to(mv[kk], (L,))
                        t = w - a0 - mb * dv
                        acc = acc + t * t
                    return acc

                return lax.fori_loop(0, CH // L, grp_body, acc)

            acc = lax.fori_loop(0, JD, col_body, acc)
            h = h_next

        accv[...] = acc
        pltpu.sync_copy(accv, out_hbm.at[wid])

    return k(w_mean, ages, anchors)


def _tc_body(w_ref, ages_ref, anch_ref, out_ref):
    i = pl.program_id(0)

    @pl.when(i == 0)
    def _():
        out_ref[...] = jnp.zeros_like(out_ref)

    ages = ages_ref[0]
    d0 = jnp.abs(ages - LO_MID)
    d1 = jnp.abs(ages - HI_MID)
    m8 = jnp.where(d1 < d0, 1.0, 0.0).astype(jnp.float32)
    n1 = jnp.sum(m8)
    ones_row = jnp.ones((1, 128), jnp.float32)
    zeros6 = jnp.zeros((6, 128), jnp.float32)
    mm = jnp.zeros((8, D), jnp.float32)
    sq = jnp.zeros((1, D), jnp.float32)
    for j in range(8):
        wj = w_ref[pl.ds(j * 128, 128), :]
        lhs_j = jnp.concatenate(
            [ones_row, m8[j:j + 1, :], zeros6], axis=0)
        mm = mm + jnp.dot(lhs_j, wj, preferred_element_type=jnp.float32)
        sq = sq + jnp.dot(ones_row, wj * wj,
                          preferred_element_type=jnp.float32)
    contrib = jnp.concatenate(
        [sq, mm[0:2, :], jnp.full((1, D), n1, jnp.float32),
         jnp.zeros((4, D), jnp.float32)], axis=0)
    out_ref[...] += contrib

    @pl.when(i == G_TC - 1)
    def _():
        o = out_ref[...]
        a0 = anch_ref[0:1, :]
        a1 = anch_ref[1:2, :]
        sqs = jnp.sum(o[0:1, :])
        sv = o[1:2, :]
        s1v = o[2:3, :]
        n1s = jnp.sum(o[3:4, :]) * (1.0 / D)
        total = (sqs
                 - 2.0 * jnp.sum((sv - s1v) * a0)
                 - 2.0 * jnp.sum(s1v * a1)
                 + (jnp.float32(N_TC) - n1s) * jnp.sum(a0 * a0)
                 + n1s * jnp.sum(a1 * a1))
        out_ref[4:5, :] = jnp.full((1, D), total, jnp.float32)


def _tc_partials(w_mean, ages, anchors):
    anch8 = jnp.concatenate(
        [anchors, jnp.zeros((6, D), jnp.float32)], axis=0)
    return pl.pallas_call(
        _tc_body,
        grid=(G_TC,),
        in_specs=[
            pl.BlockSpec((BR, D), lambda i: (i, 0)),
            pl.BlockSpec((1, 8, 128), lambda i: (i, 0, 0)),
            pl.BlockSpec((8, D), lambda i: (0, 0)),
        ],
        out_specs=pl.BlockSpec((8, D), lambda i: (0, 0)),
        out_shape=jax.ShapeDtypeStruct((8, D), jnp.float32),
    )(w_mean, ages.reshape(N // BR, 8, 128), anch8)


def kernel(w_mean, target_ages_years, anchors):
    sc_part = _sc_partials(w_mean, target_ages_years, anchors)
    tc_part = _tc_partials(w_mean, target_ages_years, anchors)
    total = tc_part[4, 0] + jnp.sum(sc_part)
    return total / jnp.float32(N * D)


# speedup vs baseline: 1.0376x; 1.0376x over previous
import functools

import jax
import jax.numpy as jnp
from jax import lax
from jax.experimental import pallas as pl
from jax.experimental.pallas import tpu as pltpu
from jax.experimental.pallas import tpu_sc as plsc

N, D = 16384, 512
LO_MID, HI_MID = 30, 60

_info = plsc.get_sparse_core_info()
NC, NS, L = _info.num_cores, _info.num_subcores, _info.num_lanes
NW = NC * NS

N_SC = 4096
N_TC = N - N_SC
RPW = N_SC // NW
CH = 64
NCH = RPW // CH
JD = D // L

BR = 1024
G_TC = N_TC // BR


def _sc_partials(w_mean, ages, anchors):
    mesh = plsc.VectorSubcoreMesh(core_axis_name="c", subcore_axis_name="s")

    @functools.partial(
        pl.kernel,
        mesh=mesh,
        out_type=jax.ShapeDtypeStruct((NW, L), jnp.float32),
        scratch_types=[
            pltpu.VMEM((2, CH, D), jnp.float32),
            pltpu.VMEM((RPW,), jnp.float32),
            pltpu.VMEM((RPW,), jnp.int32),
            pltpu.VMEM((2, D), jnp.float32),
            pltpu.VMEM((D,), jnp.float32),
            pltpu.VMEM((L,), jnp.float32),
            pltpu.SemaphoreType.DMA,
            pltpu.SemaphoreType.DMA,
        ],
    )
    def k(w_hbm, ages_hbm, anch_hbm, out_hbm,
          wbuf, mval, agev, anch, dd, accv, sem0, sem1):
        cid = lax.axis_index("c")
        sid = lax.axis_index("s")
        wid = sid * NC + cid
        base = N_TC + wid * RPW

        pltpu.sync_copy(ages_hbm.at[pl.ds(base, RPW)], agev)
        pltpu.sync_copy(anch_hbm, anch)

        def prep_m(g, carry):
            a16 = agev[pl.ds(g * L, L)]
            d0 = jnp.abs(a16 - LO_MID)
            d1 = jnp.abs(a16 - HI_MID)
            mval[pl.ds(g * L, L)] = jnp.where(d1 < d0, 1.0, 0.0).astype(jnp.float32)
            return carry

        lax.fori_loop(0, RPW // L, prep_m, 0)

        def prep_dd(j, carry):
            dd[pl.ds(j * L, L)] = anch[1, pl.ds(j * L, L)] - anch[0, pl.ds(j * L, L)]
            return carry

        lax.fori_loop(0, JD, prep_dd, 0)

        sems = (sem0, sem1)

        def start(c, b):
            return pltpu.async_copy(
                w_hbm.at[pl.ds(base + c * CH, CH)], wbuf.at[b], sems[b])

        h = start(0, 0)
        acc = jnp.zeros((L,), jnp.float32)
        for c in range(NCH):
            b = c % 2
            h_next = start(c + 1, 1 - b) if c + 1 < NCH else None
            h.wait()

            def col_body(j, acc, c=c, b=b):
                a0 = anch[0, pl.ds(j * L, L)]
                dv = dd[pl.ds(j * L, L)]

                def grp_body(g, acc):
                    mv = mval[pl.ds(c * CH + g * L, L)]
                    for kk in range(L):
                        w = wbuf[b, g * L + kk, pl.ds(j * L, L)]
                        mb = jnp.broadcast_to(mv[kk], (L,))
                        t = w - a0 - mb * dv
                        acc = acc + t * t
                    return acc

                return lax.fori_loop(0, CH // L, grp_body, acc)

            acc = lax.fori_loop(0, JD, col_body, acc)
            h = h_next

        accv[...] = acc
        pltpu.sync_copy(accv, out_hbm.at[wid])

    return k(w_mean, ages, anchors)


def _tc_body(w_ref, ages_ref, anch_ref, out_ref):
    i = pl.program_id(0)

    @pl.when(i == 0)
    def _():
        out_ref[...] = jnp.zeros_like(out_ref)

    ages = ages_ref[0]
    d0 = jnp.abs(ages - LO_MID)
    d1 = jnp.abs(ages - HI_MID)
    m8 = jnp.where(d1 < d0, 1.0, 0.0).astype(jnp.float32)
    n1 = jnp.sum(m8)
    ones_row = jnp.ones((1, 128), jnp.float32)
    zeros6 = jnp.zeros((6, 128), jnp.float32)
    mm = jnp.zeros((8, D), jnp.float32)
    sq = jnp.zeros((1, D), jnp.float32)
    for j in range(8):
        wj = w_ref[pl.ds(j * 128, 128), :]
        lhs_j = jnp.concatenate(
            [ones_row, m8[j:j + 1, :], zeros6], axis=0)
        mm = mm + jnp.dot(lhs_j, wj, preferred_element_type=jnp.float32)
        sq = sq + jnp.dot(ones_row, wj * wj,
                          preferred_element_type=jnp.float32)
    contrib = jnp.concatenate(
        [sq, mm[0:2, :], jnp.full((1, D), n1, jnp.float32),
         jnp.zeros((4, D), jnp.float32)], axis=0)
    out_ref[...] += contrib

    @pl.when(i == G_TC - 1)
    def _():
        o = out_ref[...]
        a0 = anch_ref[0:1, :]
        a1 = anch_ref[1:2, :]
        sqs = jnp.sum(o[0:1, :])
        sv = o[1:2, :]
        s1v = o[2:3, :]
        n1s = jnp.sum(o[3:4, :]) * (1.0 / D)
        total = (sqs
                 - 2.0 * jnp.sum((sv - s1v) * a0)
                 - 2.0 * jnp.sum(s1v * a1)
                 + (jnp.float32(N_TC) - n1s) * jnp.sum(a0 * a0)
                 + n1s * jnp.sum(a1 * a1))
        out_ref[4:5, :] = jnp.full((1, D), total, jnp.float32)


def _tc_partials(w_mean, ages, anchors):
    anch8 = jnp.concatenate(
        [anchors, jnp.zeros((6, D), jnp.float32)], axis=0)
    return pl.pallas_call(
        _tc_body,
        grid=(G_TC,),
        in_specs=[
            pl.BlockSpec((BR, D), lambda i: (i, 0)),
            pl.BlockSpec((1, 8, 128), lambda i: (i, 0, 0)),
            pl.BlockSpec((8, D), lambda i: (0, 0)),
        ],
        out_specs=pl.BlockSpec((8, D), lambda i: (0, 0)),
        out_shape=jax.ShapeDtypeStruct((8, D), jnp.float32),
    )(w_mean, ages.reshape(N // BR, 8, 128), anch8)


def kernel(w_mean, target_ages_years, anchors):
    sc_part = _sc_partials(w_mean, target_ages_years, anchors)
    tc_part = _tc_partials(w_mean, target_ages_years, anchors)
    total = tc_part[4, 0] + jnp.sum(sc_part)
    return total / jnp.float32(N * D)
